# Initial kernel scaffold; baseline (speedup 1.0000x reference)
import jax
import jax.numpy as jnp
from jax.experimental import pallas as pl


def kernel(nodes, neigh_idx, features, weight, pos_index, neg_index):
    raise NotImplementedError("write your pallas kernel here")


# speedup vs baseline: 2.5498x; 2.5498x over previous
import functools

import jax
import jax.numpy as jnp
from jax import lax
from jax.experimental import pallas as pl
from jax.experimental.pallas import tpu as pltpu
from jax.experimental.pallas import tpu_sc as plsc

D = 128
EMBED = 128
S = 10
NW = 32
BPW = 1568
B_PAD = NW * BPW
SELF_CHUNK = 112
TILE = 8

_mesh = plsc.VectorSubcoreMesh(core_axis_name="c", subcore_axis_name="s")


@functools.partial(
    pl.kernel,
    out_type=(
        jax.ShapeDtypeStruct((B_PAD, D), jnp.float32),
        jax.ShapeDtypeStruct((B_PAD, D), jnp.float32),
    ),
    mesh=_mesh,
    scratch_types=(
        pltpu.VMEM((SELF_CHUNK,), jnp.int32),
        pltpu.VMEM((SELF_CHUNK, D), jnp.float32),
        pltpu.VMEM((TILE * S,), jnp.int32),
        pltpu.VMEM((TILE * S, D), jnp.float32),
        pltpu.VMEM((TILE, D), jnp.float32),
        pltpu.SemaphoreType.DMA,
    ),
)
def _sc_gather(nodes_hbm, neigh_hbm, feat_hbm, self_out, mean_out,
               idx_s, rows_s, idx_n, rows_n, acc, sem):
    wid = lax.axis_index("s") * 2 + lax.axis_index("c")
    base = wid * BPW

    def self_body(i, carry):
        off = base + i * SELF_CHUNK
        pltpu.sync_copy(nodes_hbm.at[pl.ds(off, SELF_CHUNK)], idx_s)
        pltpu.async_copy(feat_hbm.at[idx_s], rows_s, sem).wait()
        pltpu.sync_copy(rows_s, self_out.at[pl.ds(off, SELF_CHUNK)])
        return carry

    lax.fori_loop(0, BPW // SELF_CHUNK, self_body, 0)

    def neigh_body(it, carry):
        off = base + it * TILE
        pltpu.sync_copy(neigh_hbm.at[pl.ds(off * S, TILE * S)], idx_n)
        pltpu.async_copy(feat_hbm.at[idx_n], rows_n, sem).wait()

        def node_body(t, c2):
            r0 = t * S
            for v in range(D // 16):
                col = pl.ds(v * 16, 16)
                s = rows_n[r0, col]
                for j in range(1, S):
                    s = s + rows_n[r0 + j, col]
                acc[t, col] = s * jnp.float32(1.0 / S)
            return c2

        lax.fori_loop(0, TILE, node_body, 0)
        pltpu.sync_copy(acc, mean_out.at[pl.ds(off, TILE)])
        return carry

    lax.fori_loop(0, BPW // TILE, neigh_body, 0)


BLK = 512
_DN = (((1,), (1,)), ((), ()))


def _tc_body(w_ref, self_ref, mean_ref, out_ref):
    w = w_ref[...]
    out = lax.dot_general(w[:, :D], self_ref[...], _DN,
                          preferred_element_type=jnp.float32)
    out = out + lax.dot_general(w[:, D:], mean_ref[...], _DN,
                                preferred_element_type=jnp.float32)
    out_ref[...] = jnp.maximum(out, 0.0)


_tc_call = pl.pallas_call(
    _tc_body,
    grid=(B_PAD // BLK,),
    in_specs=[
        pl.BlockSpec((EMBED, 2 * D), lambda i: (0, 0)),
        pl.BlockSpec((BLK, D), lambda i: (i, 0)),
        pl.BlockSpec((BLK, D), lambda i: (i, 0)),
    ],
    out_specs=pl.BlockSpec((EMBED, BLK), lambda i: (0, i)),
    out_shape=jax.ShapeDtypeStruct((EMBED, B_PAD), jnp.float32),
)


def kernel(nodes, neigh_idx, features, weight, pos_index, neg_index):
    del pos_index, neg_index
    b = nodes.shape[0]
    nodes_p = jnp.pad(nodes.astype(jnp.int32), (0, B_PAD - b))
    neigh_p = jnp.pad(neigh_idx.astype(jnp.int32).reshape(-1),
                      (0, (B_PAD - b) * S))
    self_f, mean_f = _sc_gather(nodes_p, neigh_p, features)
    out = _tc_call(weight, self_f, mean_f)
    return out[:, :b]


# speedup vs baseline: 4.1966x; 1.6458x over previous
import functools

import jax
import jax.numpy as jnp
from jax import lax
from jax.experimental import pallas as pl
from jax.experimental.pallas import tpu as pltpu
from jax.experimental.pallas import tpu_sc as plsc

D = 128
EMBED = 128
S = 10
F = S + 1
NW = 32
BPW = 1568
B_PAD = NW * BPW
TILE = 8
NT = BPW // TILE
RING = 4

_mesh = plsc.VectorSubcoreMesh(core_axis_name="c", subcore_axis_name="s")


def _accum(rows, stage):
    def node(t, c):
        r0 = t * F
        for v in range(D // 16):
            cs = pl.ds(v * 16, 16)
            stage[t, cs] = rows[r0, cs]
            nv = rows[r0 + 1, cs]
            for j in range(2, F):
                nv = nv + rows[r0 + j, cs]
            stage[t, pl.ds(D + v * 16, 16)] = nv * jnp.float32(1.0 / S)
        return c

    lax.fori_loop(0, TILE, node, 0)


@functools.partial(
    pl.kernel,
    out_type=jax.ShapeDtypeStruct((B_PAD, 2 * D), jnp.float32),
    mesh=_mesh,
    scratch_types=(
        pltpu.VMEM((NT, TILE * F), jnp.int32),
        pltpu.VMEM((TILE * F, D), jnp.float32),
        pltpu.VMEM((TILE * F, D), jnp.float32),
        pltpu.VMEM((TILE * F, D), jnp.float32),
        pltpu.VMEM((TILE * F, D), jnp.float32),
        pltpu.VMEM((TILE, 2 * D), jnp.float32),
        pltpu.VMEM((TILE, 2 * D), jnp.float32),
        pltpu.SemaphoreType.DMA,
        pltpu.SemaphoreType.DMA,
        pltpu.SemaphoreType.DMA,
        pltpu.SemaphoreType.DMA,
        pltpu.SemaphoreType.DMA,
        pltpu.SemaphoreType.DMA,
    ),
)
def _sc_gather(idx_hbm, feat_hbm, comb_out,
               idx_v, rows0, rows1, rows2, rows3, stage0, stage1,
               g0, g1, g2, g3, w0, w1):
    wid = lax.axis_index("s") * 2 + lax.axis_index("c")
    base = wid * BPW
    rows = (rows0, rows1, rows2, rows3)
    gsem = (g0, g1, g2, g3)
    stages = (stage0, stage1)
    wsems = (w0, w1)

    pltpu.sync_copy(idx_hbm.at[wid], idx_v)
    for b in range(RING):
        pltpu.async_copy(feat_hbm.at[idx_v.at[b]], rows[b], gsem[b])

    def body(k, c):
        for b in range(RING):
            i = k * RING + b
            st = stages[b % 2]
            ws = wsems[b % 2]
            pltpu.make_async_copy(feat_hbm.at[idx_v.at[i]], rows[b],
                                  gsem[b]).wait()

            def _wait_prev_write():
                pltpu.make_async_copy(
                    st, comb_out.at[pl.ds(base + (i - 2) * TILE, TILE)],
                    ws).wait()

            if b >= 2:
                _wait_prev_write()
            else:
                pl.when(k > 0)(_wait_prev_write)

            _accum(rows[b], st)
            pltpu.async_copy(st, comb_out.at[pl.ds(base + i * TILE, TILE)],
                             ws)

            def _next_gather():
                pltpu.async_copy(feat_hbm.at[idx_v.at[i + RING]], rows[b],
                                 gsem[b])

            pl.when(k < NT // RING - 1)(_next_gather)
        return c

    lax.fori_loop(0, NT // RING, body, 0)
    pltpu.make_async_copy(
        stage0, comb_out.at[pl.ds(base + (NT - 2) * TILE, TILE)], w0).wait()
    pltpu.make_async_copy(
        stage1, comb_out.at[pl.ds(base + (NT - 1) * TILE, TILE)], w1).wait()


BLK = 512
_DN = (((1,), (1,)), ((), ()))


def _tc_body(w_ref, comb_ref, out_ref):
    out = lax.dot_general(w_ref[...], comb_ref[...], _DN,
                          preferred_element_type=jnp.float32)
    out_ref[...] = jnp.maximum(out, 0.0)


def _tc_call(b):
    return pl.pallas_call(
        _tc_body,
        grid=(B_PAD // BLK,),
        in_specs=[
            pl.BlockSpec((EMBED, 2 * D), lambda i: (0, 0)),
            pl.BlockSpec((BLK, 2 * D), lambda i: (i, 0)),
        ],
        out_specs=pl.BlockSpec((EMBED, BLK), lambda i: (0, i)),
        out_shape=jax.ShapeDtypeStruct((EMBED, b), jnp.float32),
    )


def kernel(nodes, neigh_idx, features, weight, pos_index, neg_index):
    del pos_index, neg_index
    b = nodes.shape[0]
    nodes_p = jnp.pad(nodes.astype(jnp.int32), (0, B_PAD - b))
    neigh_p = jnp.pad(neigh_idx.astype(jnp.int32),
                      ((0, B_PAD - b), (0, 0)))
    idx_cat = jnp.concatenate([nodes_p[:, None], neigh_p], axis=1)
    idx3 = idx_cat.reshape(NW, NT, TILE * F)
    comb = _sc_gather(idx3, features)
    return _tc_call(b)(weight, comb)


# speedup vs baseline: 4.3537x; 1.0374x over previous
import functools

import jax
import jax.numpy as jnp
from jax import lax
from jax.experimental import pallas as pl
from jax.experimental.pallas import tpu as pltpu
from jax.experimental.pallas import tpu_sc as plsc

D = 128
EMBED = 128
S = 10
F = S + 1
NW = 32
BPW = 1568
B_PAD = NW * BPW
TILE = 8
NT = BPW // TILE
RING = 7

_mesh = plsc.VectorSubcoreMesh(core_axis_name="c", subcore_axis_name="s")


def _accum(rows, stage):
    def node(t, c):
        r0 = t * F
        for v in range(D // 16):
            cs = pl.ds(v * 16, 16)
            stage[t, cs] = rows[r0, cs]
            vals = [rows[r0 + j, cs] for j in range(1, F)]
            while len(vals) > 1:
                nxt = [vals[i] + vals[i + 1] for i in range(0, len(vals) - 1, 2)]
                if len(vals) % 2:
                    nxt.append(vals[-1])
                vals = nxt
            stage[t, pl.ds(D + v * 16, 16)] = vals[0] * jnp.float32(1.0 / S)
        return c

    lax.fori_loop(0, TILE, node, 0)


@functools.partial(
    pl.kernel,
    out_type=jax.ShapeDtypeStruct((B_PAD, 2 * D), jnp.float32),
    mesh=_mesh,
    scratch_types=(
        (pltpu.VMEM((NT, TILE * F), jnp.int32),)
        + tuple(pltpu.VMEM((TILE * F, D), jnp.float32)
                for _ in range(RING))
        + tuple(pltpu.VMEM((TILE, 2 * D), jnp.float32)
                for _ in range(RING))
        + tuple(pltpu.SemaphoreType.DMA for _ in range(2 * RING))
    ),
)
def _sc_gather(idx_hbm, feat_hbm, comb_out, idx_v, *bufs):
    wid = lax.axis_index("s") * 2 + lax.axis_index("c")
    base = wid * BPW
    rows = bufs[:RING]
    stages = bufs[RING:2 * RING]
    gsem = bufs[2 * RING:3 * RING]
    wsems = bufs[3 * RING:4 * RING]

    pltpu.sync_copy(idx_hbm.at[wid], idx_v)
    for b in range(RING):
        pltpu.async_copy(feat_hbm.at[idx_v.at[b]], rows[b], gsem[b])

    def body(k, c):
        for b in range(RING):
            i = k * RING + b
            st = stages[b]
            ws = wsems[b]
            pltpu.make_async_copy(feat_hbm.at[idx_v.at[i]], rows[b],
                                  gsem[b]).wait()

            def _wait_prev_write():
                pltpu.make_async_copy(
                    st, comb_out.at[pl.ds(base + (i - RING) * TILE, TILE)],
                    ws).wait()

            pl.when(k > 0)(_wait_prev_write)

            _accum(rows[b], st)
            pltpu.async_copy(st, comb_out.at[pl.ds(base + i * TILE, TILE)],
                             ws)

            def _next_gather():
                pltpu.async_copy(feat_hbm.at[idx_v.at[i + RING]], rows[b],
                                 gsem[b])

            pl.when(k < NT // RING - 1)(_next_gather)
        return c

    lax.fori_loop(0, NT // RING, body, 0)
    for b in range(RING):
        pltpu.make_async_copy(
            stages[b],
            comb_out.at[pl.ds(base + (NT - RING + b) * TILE, TILE)],
            wsems[b]).wait()


BLK = 512
_DN = (((1,), (1,)), ((), ()))


def _tc_body(w_ref, comb_ref, out_ref):
    out = lax.dot_general(w_ref[...], comb_ref[...], _DN,
                          preferred_element_type=jnp.float32)
    out_ref[...] = jnp.maximum(out, 0.0)


def _tc_call(b):
    return pl.pallas_call(
        _tc_body,
        grid=(B_PAD // BLK,),
        in_specs=[
            pl.BlockSpec((EMBED, 2 * D), lambda i: (0, 0)),
            pl.BlockSpec((BLK, 2 * D), lambda i: (i, 0)),
        ],
        out_specs=pl.BlockSpec((EMBED, BLK), lambda i: (0, i)),
        out_shape=jax.ShapeDtypeStruct((EMBED, b), jnp.float32),
    )


def kernel(nodes, neigh_idx, features, weight, pos_index, neg_index):
    del pos_index, neg_index
    b = nodes.shape[0]
    nodes_p = jnp.pad(nodes.astype(jnp.int32), (0, B_PAD - b))
    neigh_p = jnp.pad(neigh_idx.astype(jnp.int32),
                      ((0, B_PAD - b), (0, 0)))
    idx_cat = jnp.concatenate([nodes_p[:, None], neigh_p], axis=1)
    idx3 = idx_cat.reshape(NW, NT, TILE * F)
    comb = _sc_gather(idx3, features)
    return _tc_call(b)(weight, comb)
